# baseline (speedup 1.0000x reference)
import functools

import jax
import jax.numpy as jnp
from jax import lax
from jax.experimental import pallas as pl
from jax.experimental.pallas import tpu as pltpu
from jax.experimental.pallas import tpu_sc as plsc

N = 10000
F = 128
E = 320000

CHUNK = 80
NC, NS = 2, 16
NW = NC * NS
EP = 327680
NROWS = EP // CHUNK
CH_SPLIT = EP // NW // CHUNK
GRP = 32
NBUF = 4
N_ACC = 10128
WB = 632

BLK = 1000
NBLK = N // BLK

@functools.cache
def _mesh():
    return plsc.VectorSubcoreMesh(
        core_axis_name="c", subcore_axis_name="s", num_cores=NC, num_subcores=NS
    )


def _tile_rows(s):
    off = jnp.where(s == NS - 1, N - WB, s * WB)
    return pl.multiple_of(off, 8)


def _preload(init_hbm, acc, s):
    off = _tile_rows(s)
    pltpu.sync_copy(init_hbm.at[pl.ds(off, WB)], acc.at[pl.ds(off, WB)])


def _edge_loop(x_hbm, srcp, dstp, row0, sidx, didx, acc, rows, gsem, ssem, n_groups):
    for g in range(n_groups):
        base = row0 + g * GRP
        pltpu.sync_copy(srcp.at[pl.ds(base, GRP)], sidx)
        pltpu.sync_copy(dstp.at[pl.ds(base, GRP)], didx)

        for k in range(NBUF):
            pltpu.async_copy(x_hbm.at[sidx.at[k]], rows[k], gsem[k])

        def step(i, carry):
            for k in range(NBUF):
                jj = NBUF * i + k
                pltpu.make_async_copy(x_hbm.at[sidx.at[jj]], rows[k], gsem[k]).wait()
                pltpu.async_copy(rows[k], acc.at[didx.at[jj]], ssem[k], add=True)
            for k in range(NBUF):
                jj = NBUF * i + k
                pltpu.make_async_copy(rows[k], acc.at[didx.at[jj]], ssem[k]).wait()

                @pl.when(jj + NBUF < GRP)
                def _():
                    pltpu.async_copy(x_hbm.at[sidx.at[jj + NBUF]], rows[k], gsem[k])

            return carry

        lax.fori_loop(0, GRP // NBUF, step, 0)


def _writeback(acc, out_hbm, s):
    off = _tile_rows(s)
    pltpu.sync_copy(acc.at[pl.ds(off, WB)], out_hbm.at[pl.ds(off, WB)])


@functools.cache
def _spmm_split():
    return pl.kernel(
        _spmm_split_body,
        out_type=[
            jax.ShapeDtypeStruct((N, F), jnp.float32),
            jax.ShapeDtypeStruct((N, F), jnp.float32),
        ],
        mesh=_mesh(),
        scratch_types=[
            pltpu.VMEM_SHARED((N_ACC, F), jnp.float32),
            pltpu.VMEM((GRP, CHUNK), jnp.int32),
            pltpu.VMEM((GRP, CHUNK), jnp.int32),
        ]
        + [pltpu.VMEM((CHUNK, F), jnp.float32)] * NBUF
        + [pltpu.SemaphoreType.DMA] * (2 * NBUF),
    )


def _spmm_split_body(
    x_hbm, srcp, dstp, zeros_hbm, p0, p1, acc, sidx, didx,
    r0, r1, r2, r3, g0, g1, g2, g3, s0, s1_, s2_, s3,
):
    c = lax.axis_index("c")
    s = lax.axis_index("s")
    _preload(zeros_hbm, acc, s)
    row0 = (c * NS + s) * CH_SPLIT
    plsc.subcore_barrier()
    _edge_loop(x_hbm, srcp, dstp, row0, sidx, didx, acc,
               (r0, r1, r2, r3), (g0, g1, g2, g3), (s0, s1_, s2_, s3),
               CH_SPLIT // GRP)
    plsc.subcore_barrier()

    @pl.when(c == 0)
    def _():
        _writeback(acc, p0, s)

    @pl.when(c == 1)
    def _():
        _writeback(acc, p1, s)


def _mm_body(x_ref, w_ref, o_ref):
    o_ref[...] = jnp.dot(x_ref[...], w_ref[...], preferred_element_type=jnp.float32)


def _matmul(x, w):
    return pl.pallas_call(
        _mm_body,
        grid=(NBLK,),
        in_specs=[
            pl.BlockSpec((BLK, F), lambda i: (i, 0)),
            pl.BlockSpec((F, F), lambda i: (0, 0)),
        ],
        out_specs=pl.BlockSpec((BLK, F), lambda i: (i, 0)),
        out_shape=jax.ShapeDtypeStruct((N, F), jnp.float32),
    )(x, w)


def _wmix_body(w2, wt, l_ref, r_ref):
    l_ref[...] = jnp.dot(w2[...], wt[0:F], preferred_element_type=jnp.float32)
    r_ref[...] = jnp.dot(w2[...], wt[F : 2 * F], preferred_element_type=jnp.float32)


def _wmix(w2, wt):
    return pl.pallas_call(
        _wmix_body,
        out_shape=[
            jax.ShapeDtypeStruct((F, F), jnp.float32),
            jax.ShapeDtypeStruct((F, F), jnp.float32),
        ],
    )(w2, wt)


def _low_body(p0, p1, sup, w2l, g, b, as_out, x1_out, s1, s2, stats):
    ph = pl.program_id(0)
    j = pl.program_id(1)
    acc = p0[...] + p1[...]
    x = acc + sup[...]
    as_out[...] = acc

    @pl.when(ph == 0)
    def _():
        @pl.when(j == 0)
        def _():
            s1[...] = jnp.zeros_like(s1)
            s2[...] = jnp.zeros_like(s2)

        s1[...] += jnp.sum(x, axis=0, keepdims=True)
        s2[...] += jnp.sum(x * x, axis=0, keepdims=True)
        x1_out[...] = x

        @pl.when(j == NBLK - 1)
        def _():
            m = s1[...] / N
            v = s2[...] / N - m * m
            stats[0:1] = m
            stats[1:2] = g[...] * lax.rsqrt(v + 1e-5)

    @pl.when(ph == 1)
    def _():
        xn = (x - stats[0:1]) * stats[1:2] + b[...]
        x1_out[...] = jnp.dot(
            jnp.maximum(xn, 0.0), w2l[...], preferred_element_type=jnp.float32
        )


def _low_stage(p0, p1, sup, w2l, g, b):
    blk = pl.BlockSpec((BLK, F), lambda ph, j: (j, 0))
    full = lambda r: pl.BlockSpec(r, lambda ph, j: (0, 0))
    return pl.pallas_call(
        _low_body,
        grid=(2, NBLK),
        in_specs=[blk, blk, blk, full((F, F)), full((1, F)), full((1, F))],
        out_specs=[blk, blk],
        out_shape=[
            jax.ShapeDtypeStruct((N, F), jnp.float32),
            jax.ShapeDtypeStruct((N, F), jnp.float32),
        ],
        scratch_shapes=[
            pltpu.VMEM((1, F), jnp.float32),
            pltpu.VMEM((1, F), jnp.float32),
            pltpu.VMEM((2, F), jnp.float32),
        ],
    )(p0, p1, sup, w2l, g, b)


def _mid_body(q0, q1, sup, w2r, g, b, x1, bc, z_out, i2_out, s1, s2, stats):
    ph = pl.program_id(0)
    j = pl.program_id(1)
    x = q0[...] + q1[...] - sup[...]

    @pl.when(ph == 0)
    def _():
        @pl.when(j == 0)
        def _():
            s1[...] = jnp.zeros_like(s1)
            s2[...] = jnp.zeros_like(s2)

        s1[...] += jnp.sum(x, axis=0, keepdims=True)
        s2[...] += jnp.sum(x * x, axis=0, keepdims=True)
        z_out[...] = x
        i2_out[...] = x

        @pl.when(j == NBLK - 1)
        def _():
            m = s1[...] / N
            v = s2[...] / N - m * m
            stats[0:1] = m
            stats[1:2] = g[...] * lax.rsqrt(v + 1e-5)

    @pl.when(ph == 1)
    def _():
        xn = (x - stats[0:1]) * stats[1:2] + b[...]
        z = jnp.dot(jnp.maximum(xn, 0.0), w2r[...], preferred_element_type=jnp.float32)
        z_out[...] = z
        i2_out[...] = x1[...] - z + bc[...]


def _mid_stage(q0, q1, sup, w2r, g, b, x1, bc):
    blk = pl.BlockSpec((BLK, F), lambda ph, j: (j, 0))
    full = lambda r: pl.BlockSpec(r, lambda ph, j: (0, 0))
    return pl.pallas_call(
        _mid_body,
        grid=(2, NBLK),
        in_specs=[blk, blk, blk, full((F, F)), full((1, F)), full((1, F)), blk,
                  full((1, F))],
        out_specs=[blk, blk],
        out_shape=[
            jax.ShapeDtypeStruct((N, F), jnp.float32),
            jax.ShapeDtypeStruct((N, F), jnp.float32),
        ],
        scratch_shapes=[
            pltpu.VMEM((1, F), jnp.float32),
            pltpu.VMEM((1, F), jnp.float32),
            pltpu.VMEM((2, F), jnp.float32),
        ],
    )(q0, q1, sup, w2r, g, b, x1, bc)


def _add3_body(a, b, c, o_ref):
    o_ref[...] = a[...] + b[...] + c[...]


def _add3(a, b, c):
    blk = pl.BlockSpec((BLK, F), lambda i: (i, 0))
    return pl.pallas_call(
        _add3_body,
        grid=(NBLK,),
        in_specs=[blk, blk, blk],
        out_specs=blk,
        out_shape=jax.ShapeDtypeStruct((N, F), jnp.float32),
    )(a, b, c)


def kernel(feature, edge_index, W1, W2, g1, b1, g2, b2, Wcat, bcat):
    src = edge_index[1]
    dst = edge_index[0]
    pad = EP - E
    spread = jnp.arange(pad, dtype=jnp.int32) % CHUNK
    srcp = jnp.concatenate([src, spread]).reshape(NROWS, CHUNK)
    dstp = jnp.concatenate([dst, N + spread]).reshape(NROWS, CHUNK)
    zeros = jnp.zeros((N, F), jnp.float32)
    g1r, b1r = g1.reshape(1, F), b1.reshape(1, F)
    g2r, b2r = g2.reshape(1, F), b2.reshape(1, F)
    bc = bcat.reshape(1, F)

    spmm = _spmm_split()
    support = _matmul(feature, W1)
    w2l, w2r = _wmix(W2, Wcat.T)
    p0, p1 = spmm(support, srcp, dstp, zeros)
    As, x1 = _low_stage(p0, p1, support, w2l, g1r, b1r)
    q0, q1 = spmm(As, srcp, dstp, zeros)
    z, init2 = _mid_stage(q0, q1, support, w2r, g2r, b2r, x1, bc)
    v0, v1 = spmm(z, srcp, dstp, zeros)
    vfull = _add3(v0, v1, x1)
    o0, o1 = spmm(vfull, srcp, dstp, zeros)
    return _add3(o0, o1, init2)


# speedup vs baseline: 1.0806x; 1.0806x over previous
import functools

import jax
import jax.numpy as jnp
from jax import lax
from jax.experimental import pallas as pl
from jax.experimental.pallas import tpu as pltpu
from jax.experimental.pallas import tpu_sc as plsc

N = 10000
F = 128
E = 320000

CHUNK = 128
NC, NS = 2, 16
NW = NC * NS
EP = 327680
NROWS = EP // CHUNK
CH_SPLIT = EP // NW // CHUNK
GRP = 40
N_ACC = 10136
WB = 632

BLK = 1000
NBLK = N // BLK

@functools.cache
def _mesh():
    return plsc.VectorSubcoreMesh(
        core_axis_name="c", subcore_axis_name="s", num_cores=NC, num_subcores=NS
    )


def _tile_rows(s):
    off = jnp.where(s == NS - 1, N - WB, s * WB)
    return pl.multiple_of(off, 8)


def _preload(init_hbm, acc, s):
    off = _tile_rows(s)
    pltpu.sync_copy(init_hbm.at[pl.ds(off, WB)], acc.at[pl.ds(off, WB)])


def _edge_loop(x_hbm, srcp, dstp, row0, sidx, didx, acc, rows_a, rows_b,
               sem_ga, sem_gb, n_groups):
    half = GRP // 2
    for g in range(n_groups):
        base = row0 + g * GRP
        pltpu.sync_copy(srcp.at[pl.ds(base, GRP)], sidx)
        pltpu.sync_copy(dstp.at[pl.ds(base, GRP)], didx)

        pltpu.async_copy(x_hbm.at[sidx.at[0]], rows_a, sem_ga)

        def step2(i, carry):
            j0 = 2 * i
            j1 = j0 + 1
            pltpu.async_copy(x_hbm.at[sidx.at[j1]], rows_b, sem_gb)
            pltpu.make_async_copy(x_hbm.at[sidx.at[j0]], rows_a, sem_ga).wait()
            pltpu.sync_copy(rows_a, acc.at[didx.at[j0]], add=True)

            @pl.when(i + 1 < half)
            def _():
                pltpu.async_copy(x_hbm.at[sidx.at[j0 + 2]], rows_a, sem_ga)

            pltpu.make_async_copy(x_hbm.at[sidx.at[j1]], rows_b, sem_gb).wait()
            pltpu.sync_copy(rows_b, acc.at[didx.at[j1]], add=True)
            return carry

        lax.fori_loop(0, half, step2, 0)


def _writeback(acc, out_hbm, s):
    off = _tile_rows(s)
    pltpu.sync_copy(acc.at[pl.ds(off, WB)], out_hbm.at[pl.ds(off, WB)])


@functools.cache
def _spmm_split():
    return pl.kernel(
        _spmm_split_body,
        out_type=[
            jax.ShapeDtypeStruct((N, F), jnp.float32),
            jax.ShapeDtypeStruct((N, F), jnp.float32),
        ],
        mesh=_mesh(),
        scratch_types=[
            pltpu.VMEM_SHARED((N_ACC, F), jnp.float32),
            pltpu.VMEM((GRP, CHUNK), jnp.int32),
            pltpu.VMEM((GRP, CHUNK), jnp.int32),
            pltpu.VMEM((CHUNK, F), jnp.float32),
            pltpu.VMEM((CHUNK, F), jnp.float32),
            pltpu.SemaphoreType.DMA,
            pltpu.SemaphoreType.DMA,
        ],
    )


def _spmm_split_body(
    x_hbm, srcp, dstp, zeros_hbm, p0, p1, acc, sidx, didx, rows_a, rows_b,
    sem_ga, sem_gb,
):
    c = lax.axis_index("c")
    s = lax.axis_index("s")
    _preload(zeros_hbm, acc, s)
    row0 = (c * NS + s) * CH_SPLIT
    plsc.subcore_barrier()
    _edge_loop(x_hbm, srcp, dstp, row0, sidx, didx, acc, rows_a, rows_b,
               sem_ga, sem_gb, CH_SPLIT // GRP)
    plsc.subcore_barrier()

    @pl.when(c == 0)
    def _():
        _writeback(acc, p0, s)

    @pl.when(c == 1)
    def _():
        _writeback(acc, p1, s)


def _mm_body(x_ref, w_ref, o_ref):
    o_ref[...] = jnp.dot(x_ref[...], w_ref[...], preferred_element_type=jnp.float32)


def _matmul(x, w):
    return pl.pallas_call(
        _mm_body,
        grid=(NBLK,),
        in_specs=[
            pl.BlockSpec((BLK, F), lambda i: (i, 0)),
            pl.BlockSpec((F, F), lambda i: (0, 0)),
        ],
        out_specs=pl.BlockSpec((BLK, F), lambda i: (i, 0)),
        out_shape=jax.ShapeDtypeStruct((N, F), jnp.float32),
    )(x, w)


def _wmix_body(w2, wt, l_ref, r_ref):
    l_ref[...] = jnp.dot(w2[...], wt[0:F], preferred_element_type=jnp.float32)
    r_ref[...] = jnp.dot(w2[...], wt[F : 2 * F], preferred_element_type=jnp.float32)


def _wmix(w2, wt):
    return pl.pallas_call(
        _wmix_body,
        out_shape=[
            jax.ShapeDtypeStruct((F, F), jnp.float32),
            jax.ShapeDtypeStruct((F, F), jnp.float32),
        ],
    )(w2, wt)


def _low_body(p0, p1, sup, w2l, g, b, as_out, x1_out, s1, s2, stats):
    ph = pl.program_id(0)
    j = pl.program_id(1)
    acc = p0[...] + p1[...]
    x = acc + sup[...]
    as_out[...] = acc

    @pl.when(ph == 0)
    def _():
        @pl.when(j == 0)
        def _():
            s1[...] = jnp.zeros_like(s1)
            s2[...] = jnp.zeros_like(s2)

        s1[...] += jnp.sum(x, axis=0, keepdims=True)
        s2[...] += jnp.sum(x * x, axis=0, keepdims=True)
        x1_out[...] = x

        @pl.when(j == NBLK - 1)
        def _():
            m = s1[...] / N
            v = s2[...] / N - m * m
            stats[0:1] = m
            stats[1:2] = g[...] * lax.rsqrt(v + 1e-5)

    @pl.when(ph == 1)
    def _():
        xn = (x - stats[0:1]) * stats[1:2] + b[...]
        x1_out[...] = jnp.dot(
            jnp.maximum(xn, 0.0), w2l[...], preferred_element_type=jnp.float32
        )


def _low_stage(p0, p1, sup, w2l, g, b):
    blk = pl.BlockSpec((BLK, F), lambda ph, j: (j, 0))
    full = lambda r: pl.BlockSpec(r, lambda ph, j: (0, 0))
    return pl.pallas_call(
        _low_body,
        grid=(2, NBLK),
        in_specs=[blk, blk, blk, full((F, F)), full((1, F)), full((1, F))],
        out_specs=[blk, blk],
        out_shape=[
            jax.ShapeDtypeStruct((N, F), jnp.float32),
            jax.ShapeDtypeStruct((N, F), jnp.float32),
        ],
        scratch_shapes=[
            pltpu.VMEM((1, F), jnp.float32),
            pltpu.VMEM((1, F), jnp.float32),
            pltpu.VMEM((2, F), jnp.float32),
        ],
    )(p0, p1, sup, w2l, g, b)


def _mid_body(q0, q1, sup, w2r, g, b, x1, bc, z_out, i2_out, s1, s2, stats):
    ph = pl.program_id(0)
    j = pl.program_id(1)
    x = q0[...] + q1[...] - sup[...]

    @pl.when(ph == 0)
    def _():
        @pl.when(j == 0)
        def _():
            s1[...] = jnp.zeros_like(s1)
            s2[...] = jnp.zeros_like(s2)

        s1[...] += jnp.sum(x, axis=0, keepdims=True)
        s2[...] += jnp.sum(x * x, axis=0, keepdims=True)
        z_out[...] = x
        i2_out[...] = x

        @pl.when(j == NBLK - 1)
        def _():
            m = s1[...] / N
            v = s2[...] / N - m * m
            stats[0:1] = m
            stats[1:2] = g[...] * lax.rsqrt(v + 1e-5)

    @pl.when(ph == 1)
    def _():
        xn = (x - stats[0:1]) * stats[1:2] + b[...]
        z = jnp.dot(jnp.maximum(xn, 0.0), w2r[...], preferred_element_type=jnp.float32)
        z_out[...] = z
        i2_out[...] = x1[...] - z + bc[...]


def _mid_stage(q0, q1, sup, w2r, g, b, x1, bc):
    blk = pl.BlockSpec((BLK, F), lambda ph, j: (j, 0))
    full = lambda r: pl.BlockSpec(r, lambda ph, j: (0, 0))
    return pl.pallas_call(
        _mid_body,
        grid=(2, NBLK),
        in_specs=[blk, blk, blk, full((F, F)), full((1, F)), full((1, F)), blk,
                  full((1, F))],
        out_specs=[blk, blk],
        out_shape=[
            jax.ShapeDtypeStruct((N, F), jnp.float32),
            jax.ShapeDtypeStruct((N, F), jnp.float32),
        ],
        scratch_shapes=[
            pltpu.VMEM((1, F), jnp.float32),
            pltpu.VMEM((1, F), jnp.float32),
            pltpu.VMEM((2, F), jnp.float32),
        ],
    )(q0, q1, sup, w2r, g, b, x1, bc)


def _add3_body(a, b, c, o_ref):
    o_ref[...] = a[...] + b[...] + c[...]


def _add3(a, b, c):
    blk = pl.BlockSpec((BLK, F), lambda i: (i, 0))
    return pl.pallas_call(
        _add3_body,
        grid=(NBLK,),
        in_specs=[blk, blk, blk],
        out_specs=blk,
        out_shape=jax.ShapeDtypeStruct((N, F), jnp.float32),
    )(a, b, c)


def kernel(feature, edge_index, W1, W2, g1, b1, g2, b2, Wcat, bcat):
    src = edge_index[1]
    dst = edge_index[0]
    pad = EP - E
    spread = jnp.arange(pad, dtype=jnp.int32) % CHUNK
    srcp = jnp.concatenate([src, spread]).reshape(NROWS, CHUNK)
    dstp = jnp.concatenate([dst, N + spread]).reshape(NROWS, CHUNK)
    zeros = jnp.zeros((N, F), jnp.float32)
    g1r, b1r = g1.reshape(1, F), b1.reshape(1, F)
    g2r, b2r = g2.reshape(1, F), b2.reshape(1, F)
    bc = bcat.reshape(1, F)

    spmm = _spmm_split()
    support = _matmul(feature, W1)
    w2l, w2r = _wmix(W2, Wcat.T)
    p0, p1 = spmm(support, srcp, dstp, zeros)
    As, x1 = _low_stage(p0, p1, support, w2l, g1r, b1r)
    q0, q1 = spmm(As, srcp, dstp, zeros)
    z, init2 = _mid_stage(q0, q1, support, w2r, g2r, b2r, x1, bc)
    v0, v1 = spmm(z, srcp, dstp, zeros)
    vfull = _add3(v0, v1, x1)
    o0, o1 = spmm(vfull, srcp, dstp, zeros)
    return _add3(o0, o1, init2)
